# baseline (speedup 1.0000x reference)
import functools
import math

import jax
import jax.numpy as jnp
from jax import lax
from jax.experimental import pallas as pl
from jax.experimental.pallas import tpu as pltpu
from jax.experimental.pallas import tpu_sc as plsc

_D = 128
_NC = 2
_NS = 16
_NW = _NC * _NS
_CH = 16

_OFF_ELO = 0
_OFF_TC = 14
_OFF_URG = 17
_OFF_INC = 33
_OFF_MY = 38
_OFF_OPP = 54
_FUSED_ROWS = 72

_LOG_BIN_THRESHOLDS = (
    0.5, 1.5, 3.5, 5.5, 9.5, 15.5, 25.5, 41.5, 66.5, 107.5,
    172.5, 276.5, 442.5, 707.5, 1130.5,
)


def _fuse_body(elo_e, tc_e, urg_e, inc_e, my_e, opp_e, pos_e, out_ref):
    p = pos_e[...]
    fused = jnp.concatenate(
        [
            elo_e[...] + p[0:1],
            tc_e[...] + p[1:2],
            urg_e[...] + p[2:3],
            inc_e[...] + p[3:4],
            my_e[...] + p[4:5],
            opp_e[...] + p[5:6],
            jnp.zeros((_FUSED_ROWS - 70, _D), jnp.float32),
        ],
        axis=0,
    )
    out_ref[...] = fused


def _fuse_tables(elo_e, tc_e, urg_e, inc_e, my_e, opp_e, pos_e):
    return pl.pallas_call(
        _fuse_body,
        out_shape=jax.ShapeDtypeStruct((_FUSED_ROWS, _D), jnp.float32),
    )(elo_e, tc_e, urg_e, inc_e, my_e, opp_e, pos_e)


def _take16(vec, idx):
    dnums = lax.GatherDimensionNumbers(
        offset_dims=(), collapsed_slice_dims=(0,), start_index_map=(0,))
    return lax.gather(vec, idx[:, None], dnums, (1,),
                      mode=lax.GatherScatterMode.PROMISE_IN_BOUNDS)


def _bcast_lane(vec, lane):
    return _take16(vec, jnp.full((_CH,), lane, jnp.int32))


def _sc_body(bpw, fused, elo_h, tc_h, rem_h, inc_h, my_h, opp_h, anch_h,
             out_h, elo_v, tc_v, rem_v, inc_v, my_v, opp_v, anch_v,
             idx_v, idxh_v, dst_v, rows_v, aux_v, sem_a, sem_b):
    wid = lax.axis_index("s") * _NC + lax.axis_index("c")
    base = wid * bpw

    pltpu.sync_copy(elo_h.at[pl.ds(base, bpw)], elo_v)
    pltpu.sync_copy(tc_h.at[pl.ds(base, bpw)], tc_v)
    pltpu.sync_copy(rem_h.at[pl.ds(base, bpw)], rem_v)
    pltpu.sync_copy(inc_h.at[pl.ds(base, bpw)], inc_v)
    pltpu.sync_copy(my_h.at[pl.ds(base, bpw)], my_v)
    pltpu.sync_copy(opp_h.at[pl.ds(base, bpw)], opp_v)
    pltpu.sync_copy(anch_h, anch_v)

    anch = anch_v[...]
    iota = lax.iota(jnp.int32, _CH)
    one = jnp.ones((_CH,), jnp.int32)
    zero = jnp.zeros((_CH,), jnp.int32)

    def log_bin(x):
        b = zero
        for thr in _LOG_BIN_THRESHOLDS:
            b = b + jnp.where(x >= thr, one, zero)
        return b

    def chunk_body(c, carry):
        off = c * _CH
        elo = elo_v[pl.ds(off, _CH)]
        tcv = tc_v[pl.ds(off, _CH)]
        rem = rem_v[pl.ds(off, _CH)]
        inc = inc_v[pl.ds(off, _CH)]
        myt = my_v[pl.ds(off, _CH)]
        opp = opp_v[pl.ds(off, _CH)]

        ec = jnp.clip(elo, anch[0], anch[13])
        cnt = zero
        for k in range(14):
            cnt = cnt + jnp.where(ec >= anch[k], one, zero)
        li = jnp.clip(cnt - 1, 0, 12)
        la = _take16(anch, li)
        ua = _take16(anch, li + 1)
        t = jnp.clip((ec - la) / (ua - la + 1e-6), 0.0, 1.0)

        ub = log_bin(rem)
        mb = log_bin(myt)
        ob = log_bin(opp)
        ib = (jnp.where(inc == 1.0, one, zero)
              + 2 * jnp.where(inc == 2.0, one, zero)
              + 3 * jnp.where((inc >= 3.0) & (inc < 10.0), one, zero)
              + 4 * jnp.where(inc >= 10.0, one, zero))

        idx_v[pl.ds(0, _CH)] = li + _OFF_ELO
        idx_v[pl.ds(_CH, _CH)] = tcv + _OFF_TC
        idx_v[pl.ds(2 * _CH, _CH)] = ub + _OFF_URG
        idx_v[pl.ds(3 * _CH, _CH)] = ib + _OFF_INC
        idx_v[pl.ds(4 * _CH, _CH)] = mb + _OFF_MY
        idx_v[pl.ds(5 * _CH, _CH)] = ob + _OFF_OPP
        idxh_v[...] = li + 1

        out_base = (base + off) * 6
        for j in range(6):
            dst_v[pl.ds(j * _CH, _CH)] = out_base + iota * 6 + j

        ga = pltpu.async_copy(fused.at[idx_v], rows_v, sem_a)
        gb = pltpu.async_copy(fused.at[idxh_v], aux_v, sem_b)
        ga.wait()
        gb.wait()

        for e in range(_CH):
            te = _bcast_lane(t, e)
            for k in range(_D // _CH):
                sl = pl.ds(k * _CH, _CH)
                lo_row = rows_v[e, sl]
                hi_row = aux_v[e, sl]
                rows_v[e, sl] = lo_row + te * (hi_row - lo_row)

        pltpu.async_copy(rows_v, out_h.at[dst_v], sem_a).wait()
        return carry

    lax.fori_loop(0, bpw // _CH, chunk_body, 0)


def kernel(player_elo, tc_cat, remaining_time, increment, my_last_time,
           opp_last_time, elo_anchors, elo_embeddings, tc_embedding,
           urgency_embedding, inc_embedding, my_time_embedding,
           opp_time_embedding, token_pos_embedding):
    b = player_elo.shape[0]
    bpw = b // _NW
    tc_cat = tc_cat.astype(jnp.int32)
    anch16 = jnp.pad(elo_anchors, (0, 2))

    fused = _fuse_tables(elo_embeddings, tc_embedding, urgency_embedding,
                         inc_embedding, my_time_embedding,
                         opp_time_embedding, token_pos_embedding)

    mesh = plsc.VectorSubcoreMesh(core_axis_name="c", subcore_axis_name="s")
    sc = functools.partial(
        pl.kernel,
        out_type=jax.ShapeDtypeStruct((b * 6, _D), jnp.float32),
        mesh=mesh,
        scratch_types=[
            pltpu.VMEM((bpw,), jnp.float32),
            pltpu.VMEM((bpw,), jnp.int32),
            pltpu.VMEM((bpw,), jnp.float32),
            pltpu.VMEM((bpw,), jnp.float32),
            pltpu.VMEM((bpw,), jnp.float32),
            pltpu.VMEM((bpw,), jnp.float32),
            pltpu.VMEM((16,), jnp.float32),
            pltpu.VMEM((_CH * 6,), jnp.int32),
            pltpu.VMEM((_CH,), jnp.int32),
            pltpu.VMEM((_CH * 6,), jnp.int32),
            pltpu.VMEM((_CH * 6, _D), jnp.float32),
            pltpu.VMEM((_CH, _D), jnp.float32),
            pltpu.SemaphoreType.DMA,
            pltpu.SemaphoreType.DMA,
        ],
    )(functools.partial(_sc_body, bpw))
    out = sc(fused, player_elo, tc_cat, remaining_time, increment,
             my_last_time, opp_last_time, anch16)
    return out.reshape(b, 6, _D)
